# baseline (speedup 1.0000x reference)
import jax
import jax.numpy as jnp
from jax.experimental import pallas as pl

_E = 64
_TOP_K = 8
_TM = 2048


def _gate_kernel_t(x_ref, w_ref, b_ref, wout_ref, iout_ref):
    logits = jax.lax.dot_general(
        w_ref[...], x_ref[...],
        dimension_numbers=(((0,), (1,)), ((), ())),
        preferred_element_type=jnp.float32,
    )
    logits = logits + b_ref[...]
    m = jnp.max(logits, axis=0, keepdims=True)
    e = jnp.exp(logits - m)
    s = jnp.sum(e, axis=0, keepdims=True)
    scores = e / s
    idx = jax.lax.broadcasted_iota(jnp.int32, scores.shape, 0)
    cur = scores
    ws = []
    inds = []
    for _ in range(_TOP_K):
        mk = jnp.max(cur, axis=0, keepdims=True)
        is_max = cur == mk
        ik = jnp.min(jnp.where(is_max, idx, _E), axis=0, keepdims=True)
        ws.append(mk)
        inds.append(ik)
        cur = jnp.where(idx == ik, -1.0, cur)
    wout_ref[...] = jnp.concatenate(ws, axis=0).T
    iout_ref[...] = jnp.concatenate(inds, axis=0).T


@jax.jit
def kernel(x, W, b):
    B, S, D = x.shape
    T = B * S
    x2 = x.reshape(T, D)
    b2 = b.reshape(_E, 1)
    grid = (T // _TM,)
    weights_t, indices_t = pl.pallas_call(
        _gate_kernel_t,
        grid=grid,
        in_specs=[
            pl.BlockSpec((_TM, D), lambda i: (i, 0)),
            pl.BlockSpec((D, _E), lambda i: (0, 0)),
            pl.BlockSpec((_E, 1), lambda i: (0, 0)),
        ],
        out_specs=[
            pl.BlockSpec((_TM, _TOP_K), lambda i: (i, 0)),
            pl.BlockSpec((_TM, _TOP_K), lambda i: (i, 0)),
        ],
        out_shape=[
            jax.ShapeDtypeStruct((T, _TOP_K), jnp.float32),
            jax.ShapeDtypeStruct((T, _TOP_K), jnp.int32),
        ],
    )(x2, W, b2)
    weights = weights_t.reshape(B, S, _TOP_K)
    indices = indices_t.reshape(B, S, _TOP_K)
    return weights, indices


# speedup vs baseline: 1.3351x; 1.3351x over previous
import jax
import jax.numpy as jnp
from jax.experimental import pallas as pl
from jax.experimental.pallas import tpu as pltpu

_E = 64
_TOP_K = 8
_TM = 2048


def _gate_kernel_t(x_ref, w_ref, b_ref, wout_ref, iout_ref):
    logits = jax.lax.dot_general(
        w_ref[...], x_ref[...],
        dimension_numbers=(((0,), (1,)), ((), ())),
        preferred_element_type=jnp.float32,
    )
    logits = logits + b_ref[...]
    m = jnp.max(logits, axis=0, keepdims=True)
    e = jnp.exp(logits - m)
    s = jnp.sum(e, axis=0, keepdims=True)
    scores = e / s
    idx = jax.lax.broadcasted_iota(jnp.int32, scores.shape, 0)
    cur = scores
    ws = []
    inds = []
    for _ in range(_TOP_K):
        mk = jnp.max(cur, axis=0, keepdims=True)
        is_max = cur == mk
        ik = jnp.min(jnp.where(is_max, idx, _E), axis=0, keepdims=True)
        ws.append(mk)
        inds.append(ik)
        cur = jnp.where(idx == ik, -1.0, cur)
    wout_ref[...] = jnp.concatenate(ws, axis=0)
    iout_ref[...] = jnp.concatenate(inds, axis=0)


@jax.jit
def kernel(x, W, b):
    B, S, D = x.shape
    T = B * S
    x2 = x.reshape(T, D)
    b2 = b.reshape(_E, 1)
    grid = (T // _TM,)
    weights_t, indices_t = pl.pallas_call(
        _gate_kernel_t,
        grid=grid,
        in_specs=[
            pl.BlockSpec((_TM, D), lambda i: (i, 0)),
            pl.BlockSpec((D, _E), lambda i: (0, 0)),
            pl.BlockSpec((_E, 1), lambda i: (0, 0)),
        ],
        out_specs=[
            pl.BlockSpec((_TOP_K, _TM), lambda i: (0, i)),
            pl.BlockSpec((_TOP_K, _TM), lambda i: (0, i)),
        ],
        out_shape=[
            jax.ShapeDtypeStruct((_TOP_K, T), jnp.float32),
            jax.ShapeDtypeStruct((_TOP_K, T), jnp.int32),
        ],
        compiler_params=pltpu.CompilerParams(
            dimension_semantics=("parallel",),
        ),
    )(x2, W, b2)
    weights = weights_t.T.reshape(B, S, _TOP_K)
    indices = indices_t.T.reshape(B, S, _TOP_K)
    return weights, indices


# speedup vs baseline: 1.3354x; 1.0002x over previous
import jax
import jax.numpy as jnp
from jax.experimental import pallas as pl

_E = 64
_TOP_K = 8
_TM = 2048


def _gate_kernel_t(xa_ref, xb_ref, w_ref, b_ref, wout_ref, iout_ref):
    x_full = jnp.concatenate([xa_ref[...], xb_ref[...]], axis=1)
    logits = jax.lax.dot_general(
        w_ref[...], x_full,
        dimension_numbers=(((0,), (1,)), ((), ())),
        preferred_element_type=jnp.float32,
    )
    logits = logits + b_ref[...]
    m = jnp.max(logits, axis=0, keepdims=True)
    e = jnp.exp(logits - m)
    s = jnp.sum(e, axis=0, keepdims=True)
    scores = e / s
    idx = jax.lax.broadcasted_iota(jnp.int32, scores.shape, 0)
    cur = scores
    ws = []
    inds = []
    for _ in range(_TOP_K):
        mk = jnp.max(cur, axis=0, keepdims=True)
        is_max = cur == mk
        ik = jnp.min(jnp.where(is_max, idx, _E), axis=0, keepdims=True)
        ws.append(mk)
        inds.append(ik)
        cur = jnp.where(idx == ik, -1.0, cur)
    wout_ref[...] = jnp.concatenate(ws, axis=0)
    iout_ref[...] = jnp.concatenate(inds, axis=0)


@jax.jit
def kernel(x, W, b):
    B, S, D = x.shape
    T = B * S
    x2 = x.reshape(T, D)
    b2 = b.reshape(_E, 1)
    grid = (T // _TM,)
    weights_t, indices_t = pl.pallas_call(
        _gate_kernel_t,
        grid=grid,
        in_specs=[
            pl.BlockSpec((_TM, D // 2), lambda i: (i, 0)),
            pl.BlockSpec((_TM, D // 2), lambda i: (i, 1)),
            pl.BlockSpec((D, _E), lambda i: (0, 0)),
            pl.BlockSpec((_E, 1), lambda i: (0, 0)),
        ],
        out_specs=[
            pl.BlockSpec((_TOP_K, _TM), lambda i: (0, i)),
            pl.BlockSpec((_TOP_K, _TM), lambda i: (0, i)),
        ],
        out_shape=[
            jax.ShapeDtypeStruct((_TOP_K, T), jnp.float32),
            jax.ShapeDtypeStruct((_TOP_K, T), jnp.int32),
        ],
    )(x2, x2, W, b2)
    weights = weights_t.T.reshape(B, S, _TOP_K)
    indices = indices_t.T.reshape(B, S, _TOP_K)
    return weights, indices
